# Initial kernel scaffold; baseline (speedup 1.0000x reference)
import jax
import jax.numpy as jnp
from jax.experimental import pallas as pl


def kernel(hidden_states, ln1_w, q_w, k_w, v_w, o_w, q_norm_w, k_norm_w, ln2_w, gate_w, Wg, Wu, Wd):
    raise NotImplementedError("write your pallas kernel here")


# speedup vs baseline: 1.1706x; 1.1706x over previous
import jax
import jax.numpy as jnp
from jax.experimental import pallas as pl
from jax.experimental.pallas import tpu as pltpu

B, S, H = 1, 2048, 2048
NH, KVH, HD = 16, 8, 128
E, K, I = 8, 2, 768
EPS = 1e-6
THETA = 1e6

SB = 256
NSB = S // SB
TILE = 128
NT = (K * S) // TILE + E
P = NT * TILE

_BF = jnp.bfloat16
_F32 = jnp.float32


def _rms(x, w):
    return x * jax.lax.rsqrt(jnp.mean(x * x, axis=-1, keepdims=True) + EPS) * w


def _rot(x):
    h = HD // 2
    return jnp.concatenate([-x[:, h:], x[:, :h]], axis=-1)


def _qkv_body(x_ref, lnw_ref, w_ref, out_ref):
    xn = _rms(x_ref[...], lnw_ref[...])
    out_ref[...] = jnp.dot(xn.astype(_BF), w_ref[...],
                           preferred_element_type=_F32).astype(_BF)


def _qkv_call(x, ln1_w, wqkv):
    return pl.pallas_call(
        _qkv_body,
        grid=(NSB,),
        in_specs=[
            pl.BlockSpec((SB, H), lambda i: (i, 0)),
            pl.BlockSpec((1, H), lambda i: (0, 0)),
            pl.BlockSpec((H, 4 * H // 2), lambda i: (0, 0)),
        ],
        out_specs=pl.BlockSpec((SB, 4 * H // 2), lambda i: (i, 0)),
        out_shape=jax.ShapeDtypeStruct((S, 4 * H // 2), _BF),
        compiler_params=pltpu.CompilerParams(
            dimension_semantics=("arbitrary",)),
    )(x, ln1_w, wqkv)


def _attn_body(q_ref, k_ref, v_ref, cq_ref, sq_ref, ck_ref, sk_ref,
               qnw_ref, knw_ref, o_ref, kp_ref):
    i = pl.program_id(1)

    @pl.when(i == 0)
    def _prep_k():
        k = _rms(k_ref[...].astype(_F32), knw_ref[...])
        kp_ref[...] = (k * ck_ref[...] + _rot(k) * sk_ref[...]).astype(_BF)

    q = _rms(q_ref[...].astype(_F32), qnw_ref[...])
    q = q * cq_ref[...] + _rot(q) * sq_ref[...]
    s = jax.lax.dot_general(q.astype(_BF), kp_ref[...],
                            (((1,), (1,)), ((), ())),
                            preferred_element_type=_F32) * (HD ** -0.5)
    row = i * SB + jax.lax.broadcasted_iota(jnp.int32, (SB, S), 0)
    col = jax.lax.broadcasted_iota(jnp.int32, (SB, S), 1)
    s = jnp.where(col <= row, s, -1e30)
    m = jnp.max(s, axis=-1, keepdims=True)
    p = jnp.exp(s - m)
    p = p / jnp.sum(p, axis=-1, keepdims=True)
    o_ref[...] = jnp.dot(p.astype(_BF), v_ref[...],
                         preferred_element_type=_F32).astype(_BF)


def _attn_call(qkv, cos, sin, q_norm_w, k_norm_w):
    kvo = S // HD
    vvo = kvo + KVH
    return pl.pallas_call(
        _attn_body,
        grid=(NH, NSB),
        in_specs=[
            pl.BlockSpec((SB, HD), lambda h, i: (i, h)),
            pl.BlockSpec((S, HD), lambda h, i: (0, kvo + h // 2)),
            pl.BlockSpec((S, HD), lambda h, i: (0, vvo + h // 2)),
            pl.BlockSpec((SB, HD), lambda h, i: (i, 0)),
            pl.BlockSpec((SB, HD), lambda h, i: (i, 0)),
            pl.BlockSpec((S, HD), lambda h, i: (0, 0)),
            pl.BlockSpec((S, HD), lambda h, i: (0, 0)),
            pl.BlockSpec((1, HD), lambda h, i: (0, 0)),
            pl.BlockSpec((1, HD), lambda h, i: (0, 0)),
        ],
        out_specs=pl.BlockSpec((SB, HD), lambda h, i: (i, h)),
        out_shape=jax.ShapeDtypeStruct((S, NH * HD), _BF),
        scratch_shapes=[pltpu.VMEM((S, HD), _BF)],
        compiler_params=pltpu.CompilerParams(
            dimension_semantics=("parallel", "arbitrary")),
    )(qkv, qkv, qkv, cos, sin, cos, sin, q_norm_w, k_norm_w)


def _oproj_body(o_ref, ow_ref, x_ref, ln2_ref, gw_ref, x2_ref, h2_ref, rl_ref):
    att = jnp.dot(o_ref[...], ow_ref[...], preferred_element_type=_F32)
    x2 = x_ref[...] + att
    x2_ref[...] = x2
    h2 = _rms(x2, ln2_ref[...])
    h2_ref[...] = h2.astype(_BF)
    rl_ref[...] = jnp.dot(h2, gw_ref[...], preferred_element_type=_F32)


def _oproj_call(o, ow, x, ln2_w, gw_pad):
    return pl.pallas_call(
        _oproj_body,
        grid=(NSB,),
        in_specs=[
            pl.BlockSpec((SB, NH * HD), lambda i: (i, 0)),
            pl.BlockSpec((NH * HD, H), lambda i: (0, 0)),
            pl.BlockSpec((SB, H), lambda i: (i, 0)),
            pl.BlockSpec((1, H), lambda i: (0, 0)),
            pl.BlockSpec((H, 128), lambda i: (0, 0)),
        ],
        out_specs=[
            pl.BlockSpec((SB, H), lambda i: (i, 0)),
            pl.BlockSpec((SB, H), lambda i: (i, 0)),
            pl.BlockSpec((SB, 128), lambda i: (i, 0)),
        ],
        out_shape=[
            jax.ShapeDtypeStruct((S, H), _F32),
            jax.ShapeDtypeStruct((S, H), _BF),
            jax.ShapeDtypeStruct((S, 128), _F32),
        ],
        compiler_params=pltpu.CompilerParams(
            dimension_semantics=("arbitrary",)),
    )(o, ow, x, ln2_w, gw_pad)


def _moe_body(te_ref, xg_ref, wg_ref, wu_ref, wd_ref, wrow_ref, y_ref):
    xb = xg_ref[...]
    g = jnp.dot(xb, wg_ref[0], preferred_element_type=_F32)
    u = jnp.dot(xb, wu_ref[0], preferred_element_type=_F32)
    a = (g * jax.lax.logistic(g)) * u
    a = a * wrow_ref[:, 0:1]
    y_ref[...] = jnp.dot(a.astype(_BF), wd_ref[0], preferred_element_type=_F32)


def _moe_call(tile_expert, xg, wg, wu, wd, wrow):
    grid_spec = pltpu.PrefetchScalarGridSpec(
        num_scalar_prefetch=1,
        grid=(NT,),
        in_specs=[
            pl.BlockSpec((TILE, H), lambda t, te: (t, 0)),
            pl.BlockSpec((1, H, I), lambda t, te: (te[t], 0, 0)),
            pl.BlockSpec((1, H, I), lambda t, te: (te[t], 0, 0)),
            pl.BlockSpec((1, I, H), lambda t, te: (te[t], 0, 0)),
            pl.BlockSpec((TILE, 128), lambda t, te: (t, 0)),
        ],
        out_specs=pl.BlockSpec((TILE, H), lambda t, te: (t, 0)),
    )
    return pl.pallas_call(
        _moe_body,
        grid_spec=grid_spec,
        out_shape=jax.ShapeDtypeStruct((P, H), _F32),
        compiler_params=pltpu.CompilerParams(
            dimension_semantics=("arbitrary",)),
    )(tile_expert, xg, wg, wu, wd, wrow)


def kernel(hidden_states, ln1_w, q_w, k_w, v_w, o_w, q_norm_w, k_norm_w,
           ln2_w, gate_w, Wg, Wu, Wd):
    x = hidden_states.reshape(S, H)
    wqkv = jnp.concatenate([q_w, k_w, v_w], axis=1).astype(_BF)
    qkv = _qkv_call(x, ln1_w.reshape(1, H), wqkv)

    pos = jnp.arange(S, dtype=_F32)
    inv = 1.0 / (THETA ** (jnp.arange(0, HD, 2, dtype=_F32) / HD))
    f = pos[:, None] * inv[None, :]
    emb = jnp.concatenate([f, f], axis=-1)
    cos, sin = jnp.cos(emb), jnp.sin(emb)

    o = _attn_call(qkv, cos, sin, q_norm_w.reshape(1, HD),
                   k_norm_w.reshape(1, HD))

    gw_pad = jnp.zeros((H, 128), _F32).at[:, :E].set(gate_w)
    x2, h2b, rlp = _oproj_call(o, o_w.astype(_BF), x, ln2_w.reshape(1, H),
                               gw_pad)
    rl = rlp[:, :E]

    probs = jax.nn.softmax(rl, axis=-1)
    tw, ti = jax.lax.top_k(probs, K)
    tw = tw / jnp.sum(tw, axis=-1, keepdims=True)
    fe = ti.reshape(-1)
    fw = tw.reshape(-1).astype(_F32)
    ft = (jnp.arange(S * K, dtype=jnp.int32) // K)

    oh = (fe[:, None] == jnp.arange(E, dtype=jnp.int32)[None, :]).astype(jnp.int32)
    csum = jnp.cumsum(oh, axis=0)
    gsz = csum[-1]
    rank = jnp.take_along_axis(csum, fe[:, None], axis=1)[:, 0] - 1
    nt_e = (gsz + TILE - 1) // TILE
    tile_off = jnp.concatenate(
        [jnp.zeros((1,), jnp.int32), jnp.cumsum(nt_e)[:-1].astype(jnp.int32)])
    pos_flat = tile_off[fe] * TILE + rank

    tid_pad = jnp.zeros((P,), jnp.int32).at[pos_flat].set(ft)
    w_pad = jnp.zeros((P,), _F32).at[pos_flat].set(fw)
    tile_expert = jnp.minimum(
        jnp.searchsorted(jnp.cumsum(nt_e), jnp.arange(NT, dtype=jnp.int32),
                         side="right"),
        E - 1).astype(jnp.int32)

    xg = jnp.take(h2b, tid_pad, axis=0)
    wrow = jnp.broadcast_to(w_pad[:, None], (P, 128))

    y = _moe_call(tile_expert, xg, Wg.astype(_BF), Wu.astype(_BF),
                  Wd.astype(_BF), wrow)

    pf = pos_flat.reshape(S, K)
    out = x2 + jnp.take(y, pf[:, 0], axis=0) + jnp.take(y, pf[:, 1], axis=0)
    return out.reshape(B, S, H), rl
